# baseline (speedup 1.0000x reference)
import functools

import jax
import jax.numpy as jnp
from jax import lax
from jax.experimental import pallas as pl
from jax.experimental.pallas import tpu as pltpu
from jax.experimental.pallas import tpu_sc as plsc

_CHUNK = 64


def kernel(x, indices, emb_weight):
    B, S, D = x.shape
    rows = B * S
    xf = x.reshape(rows, D)
    idx_flat = jnp.tile(indices, B)
    emb = emb_weight.reshape(D)

    info = plsc.get_sparse_core_info()
    NC, NS, L = info.num_cores, info.num_subcores, info.num_lanes
    NW = NC * NS
    rows_w = rows // NW

    mesh = plsc.VectorSubcoreMesh(core_axis_name="c", subcore_axis_name="s")

    @functools.partial(
        pl.kernel,
        mesh=mesh,
        out_type=jax.ShapeDtypeStruct((rows, D), jnp.float32),
        scratch_types=[
            pltpu.VMEM((rows_w,), jnp.int32),
            pltpu.VMEM((D,), jnp.float32),
            pltpu.VMEM((_CHUNK, D), jnp.float32),
        ],
    )
    def k(x_hbm, idx_hbm, emb_hbm, out_hbm, idx_v, emb_v, xv):
        wid = lax.axis_index("s") * NC + lax.axis_index("c")
        base = wid * rows_w
        pltpu.sync_copy(idx_hbm.at[pl.ds(base, rows_w)], idx_v)
        pltpu.sync_copy(emb_hbm, emb_v)

        def chunk_body(c, carry):
            row0 = base + c * _CHUNK
            pltpu.sync_copy(x_hbm.at[pl.ds(row0, _CHUNK)], xv)

            def group_body(g, carry):
                idx_vec = idx_v[pl.ds(c * _CHUNK + g * L, L)]
                one_v = jnp.full((L,), 1, jnp.int32)
                scales = (idx_vec - one_v).astype(jnp.float32)

                def row_body(rr, carry):
                    r = g * L + rr
                    sel = jnp.full((L, 1), rr, jnp.int32)
                    scale = lax.gather(
                        scales,
                        sel,
                        dimension_numbers=lax.GatherDimensionNumbers(
                            offset_dims=(),
                            collapsed_slice_dims=(0,),
                            start_index_map=(0,),
                        ),
                        slice_sizes=(1,),
                        mode=lax.GatherScatterMode.PROMISE_IN_BOUNDS,
                    )

                    def col_body(j, carry):
                        sl = pl.ds(j * L, L)
                        xv[r, sl] = xv[r, sl] + scale * emb_v[sl]
                        return carry

                    return lax.fori_loop(0, D // L, col_body, carry)

                return lax.fori_loop(0, L, row_body, carry)

            carry = lax.fori_loop(0, _CHUNK // L, group_body, carry)
            pltpu.sync_copy(xv, out_hbm.at[pl.ds(row0, _CHUNK)])
            return carry

        lax.fori_loop(0, rows_w // _CHUNK, chunk_body, 0)

    out = k(xf, idx_flat, emb)
    return out.reshape(B, S, D)


# speedup vs baseline: 7.7005x; 7.7005x over previous
import jax
import jax.numpy as jnp
from jax.experimental import pallas as pl
from jax.experimental.pallas import tpu as pltpu

_ROW_BLOCK = 3840


def _body(idx_ref, emb_ref, x_ref, o_ref):
    scale = (idx_ref[0, 0, :] - 1).astype(jnp.float32)
    o_ref[...] = x_ref[...] + scale[:, None] * emb_ref[0][None, :]


def kernel(x, indices, emb_weight):
    B, S, D = x.shape
    rows = B * S
    xf = x.reshape(rows, D)
    nb = pl.cdiv(rows, _ROW_BLOCK)
    idx_flat = jnp.tile(indices, B)
    idx_pad = jnp.pad(idx_flat, (0, nb * _ROW_BLOCK - rows))
    idx3 = idx_pad.reshape(nb, 1, _ROW_BLOCK)
    out = pl.pallas_call(
        _body,
        grid=(nb,),
        in_specs=[
            pl.BlockSpec((1, 1, _ROW_BLOCK), lambda i: (i, 0, 0)),
            pl.BlockSpec((1, D), lambda i: (0, 0)),
            pl.BlockSpec((_ROW_BLOCK, D), lambda i: (i, 0)),
        ],
        out_specs=pl.BlockSpec((_ROW_BLOCK, D), lambda i: (i, 0)),
        out_shape=jax.ShapeDtypeStruct((rows, D), x.dtype),
        compiler_params=pltpu.CompilerParams(
            dimension_semantics=("parallel",),
            vmem_limit_bytes=63 * 1024 * 1024,
            allow_input_fusion=[True, False, False],
        ),
    )(idx3, emb_weight, xf)
    return out.reshape(B, S, D)


# speedup vs baseline: 7.7458x; 1.0059x over previous
import jax
import jax.numpy as jnp
from jax.experimental import pallas as pl
from jax.experimental.pallas import tpu as pltpu

_ROW_BLOCK = 3712


def _body(idx_ref, emb_ref, x_ref, o_ref):
    scale = (idx_ref[0, 0, :] - 1).astype(jnp.float32)
    o_ref[...] = x_ref[...] + scale[:, None] * emb_ref[0][None, :]


def kernel(x, indices, emb_weight):
    B, S, D = x.shape
    rows = B * S
    xf = x.reshape(rows, D)
    nb = pl.cdiv(rows, _ROW_BLOCK)
    idx_flat = jnp.tile(indices, B)
    idx_pad = jnp.pad(idx_flat, (0, nb * _ROW_BLOCK - rows))
    idx3 = idx_pad.reshape(nb, 1, _ROW_BLOCK)
    out = pl.pallas_call(
        _body,
        grid=(nb,),
        in_specs=[
            pl.BlockSpec((1, 1, _ROW_BLOCK), lambda i: (i, 0, 0)),
            pl.BlockSpec((1, D), lambda i: (0, 0)),
            pl.BlockSpec((_ROW_BLOCK, D), lambda i: (i, 0)),
        ],
        out_specs=pl.BlockSpec((_ROW_BLOCK, D), lambda i: (i, 0)),
        out_shape=jax.ShapeDtypeStruct((rows, D), x.dtype),
        compiler_params=pltpu.CompilerParams(
            dimension_semantics=("parallel",),
            vmem_limit_bytes=63 * 1024 * 1024,
            allow_input_fusion=[True, False, False],
        ),
    )(idx3, emb_weight, xf)
    return out.reshape(B, S, D)
